# baseline (speedup 1.0000x reference)
import jax
import jax.numpy as jnp
from jax import lax
from jax.experimental import pallas as pl
from jax.experimental.pallas import tpu as pltpu
from jax.experimental.pallas import tpu_sc as plsc

_BBOX = 1000
_VOCAB = 1004
_H = 128
_N = 1024 * 200
_NW = 32
_PER_W = _N // _NW
_B = 64
_NCH = 4
_NGRP = _PER_W // (_NCH * _B)
_NT = 10


def _sc_body(boxes_hbm, tab_hbm, out_hbm, bx_v, idx_v, acc_v, tab_sp, sems,
             osems):
    nc = 2
    sid = lax.axis_index("s")
    wid = sid * nc + lax.axis_index("c")

    @pl.when(sid == 0)
    def _():
        pltpu.sync_copy(tab_hbm, tab_sp)

    plsc.subcore_barrier()

    def load_boxes(p, buf):
        base = wid * _PER_W + (_NCH * p) * _B
        pltpu.sync_copy(boxes_hbm.at[:, pl.ds(base, _NCH * _B)],
                        bx_v.at[buf])

    def compute_idx(buf, s):
        for g in range(_B // 16):
            sl = pl.ds(g * 16, 16)
            bsl = pl.ds(s * _B + g * 16, 16)
            cx = bx_v[buf, 0, bsl]
            cy = bx_v[buf, 1, bsl]
            w = bx_v[buf, 2, bsl]
            h = bx_v[buf, 3, bsl]
            xs = bx_v[buf, 4, bsl]
            ys = bx_v[buf, 5, bsl]

            txs = xs - 500
            tys = ys - 500
            xsa = (txs + ((txs >> 31) & 1)) >> 1
            ysa = (tys + ((tys >> 31) & 1)) >> 1
            hw = w >> 1
            hh = h >> 1
            x1 = jnp.clip(cx - hw - xsa, 0, _BBOX)
            y1 = jnp.clip(cy - hh - ysa, 0, _BBOX)
            x3 = jnp.clip(cx + hw + xsa, 0, _BBOX)
            y3 = jnp.clip(cy + hh + ysa, 0, _BBOX)

            idx_v[buf, 0, s, sl] = w
            idx_v[buf, 1, s, sl] = h + _VOCAB
            idx_v[buf, 2, s, sl] = cx + 2 * _VOCAB
            idx_v[buf, 3, s, sl] = cy + 3 * _VOCAB
            idx_v[buf, 4, s, sl] = xs + 4 * _VOCAB
            idx_v[buf, 5, s, sl] = ys + 5 * _VOCAB
            idx_v[buf, 6, s, sl] = x1 + 6 * _VOCAB
            idx_v[buf, 7, s, sl] = y1 + 7 * _VOCAB
            idx_v[buf, 8, s, sl] = x3 + 8 * _VOCAB
            idx_v[buf, 9, s, sl] = y3 + 9 * _VOCAB

    def prep(p, buf):
        load_boxes(p, buf)
        for s in range(_NCH):
            compute_idx(buf, s)

    prep(0, 0)

    def drain_outs():
        for s in range(_NCH):
            pltpu.make_async_copy(acc_v.at[s], out_hbm.at[pl.ds(0, _B)],
                                  osems[s]).wait()

    def group(p, carry):
        buf = lax.rem(p, 2)
        base = wid * _PER_W + (_NCH * p) * _B

        @pl.when(p > 0)
        def _():
            drain_outs()

        ds_ = [pltpu.async_copy(tab_sp.at[idx_v.at[buf].at[0].at[s]],
                                acc_v.at[s], sems[s]) for s in range(_NCH)]

        prep(jnp.minimum(p + 1, _NGRP - 1), 1 - buf)

        for t in range(1, _NT):
            for s in range(_NCH):
                ds_[s].wait()
                ds_[s] = pltpu.async_copy(
                    tab_sp.at[idx_v.at[buf].at[t].at[s]], acc_v.at[s],
                    sems[s], add=True)
        for s in range(_NCH):
            ds_[s].wait()
            pltpu.async_copy(acc_v.at[s], out_hbm.at[pl.ds(base + s * _B, _B)],
                             osems[s])
        return carry

    lax.fori_loop(0, _NGRP, group, 0)
    drain_outs()


def kernel(boxes, w_embed, h_embed, cx_embed, cy_embed, xskew_embed,
           yskew_embed, x1_embed, y1_embed, x3_embed, y3_embed,
           xspecial_embed):
    boxes_t = boxes.astype(jnp.int32).reshape(_N, 6).T

    cx2 = cx_embed + xspecial_embed[0][None, :]
    cx2 = cx2.at[0].add(xspecial_embed[1] - xspecial_embed[0])
    tab = jnp.concatenate(
        [w_embed, h_embed, cx2, cy_embed, xskew_embed, yskew_embed,
         x1_embed, y1_embed, x3_embed, y3_embed], axis=0) * 0.1

    mesh = plsc.VectorSubcoreMesh(core_axis_name="c", subcore_axis_name="s")
    out = pl.kernel(
        _sc_body,
        out_type=jax.ShapeDtypeStruct((_N, _H), jnp.float32),
        mesh=mesh,
        scratch_types=[
            pltpu.VMEM((2, 6, _NCH * _B), jnp.int32),
            pltpu.VMEM((2, _NT, _NCH, _B), jnp.int32),
            pltpu.VMEM((_NCH, _B, _H), jnp.float32),
            pltpu.VMEM_SHARED((_NT * _VOCAB, _H), jnp.float32),
            [pltpu.SemaphoreType.DMA] * _NCH,
            [pltpu.SemaphoreType.DMA] * _NCH,
        ],
    )(boxes_t, tab)

    return out.reshape(boxes.shape[0], boxes.shape[1], _H)


# speedup vs baseline: 1.0242x; 1.0242x over previous
import jax
import jax.numpy as jnp
from jax import lax
from jax.experimental import pallas as pl
from jax.experimental.pallas import tpu as pltpu
from jax.experimental.pallas import tpu_sc as plsc

_BBOX = 1000
_VOCAB = 1004
_H = 128
_N = 1024 * 200
_NW = 32
_PER_W = _N // _NW
_B = 128
_NCHUNK = _PER_W // _B
_NPAIR = _NCHUNK // 2
_NT = 10


def _sc_body(boxes_hbm, tab_hbm, out_hbm, bx_v, idx_v, acc_v, tab_sp, sem_a,
             sem_b, osem_a, osem_b):
    nc = 2
    sid = lax.axis_index("s")
    wid = sid * nc + lax.axis_index("c")

    @pl.when(sid == 0)
    def _():
        pltpu.sync_copy(tab_hbm, tab_sp)

    plsc.subcore_barrier()

    def load_boxes(p, buf):
        base = wid * _PER_W + (2 * p) * _B
        pltpu.sync_copy(boxes_hbm.at[:, pl.ds(base, _B)], bx_v.at[2 * buf])
        pltpu.sync_copy(boxes_hbm.at[:, pl.ds(base + _B, _B)],
                        bx_v.at[2 * buf + 1])

    def compute_idx(buf, s):
        for g in range(_B // 16):
            sl = pl.ds(g * 16, 16)
            cx = bx_v[2 * buf + s, 0, sl]
            cy = bx_v[2 * buf + s, 1, sl]
            w = bx_v[2 * buf + s, 2, sl]
            h = bx_v[2 * buf + s, 3, sl]
            xs = bx_v[2 * buf + s, 4, sl]
            ys = bx_v[2 * buf + s, 5, sl]

            txs = xs - 500
            tys = ys - 500
            xsa = (txs + ((txs >> 31) & 1)) >> 1
            ysa = (tys + ((tys >> 31) & 1)) >> 1
            hw = w >> 1
            hh = h >> 1
            x1 = jnp.clip(cx - hw - xsa, 0, _BBOX)
            y1 = jnp.clip(cy - hh - ysa, 0, _BBOX)
            x3 = jnp.clip(cx + hw + xsa, 0, _BBOX)
            y3 = jnp.clip(cy + hh + ysa, 0, _BBOX)

            idx_v[buf, s, sl] = w
            idx_v[buf + 2, s, sl] = h + _VOCAB
            idx_v[buf + 4, s, sl] = cx + 2 * _VOCAB
            idx_v[buf + 6, s, sl] = cy + 3 * _VOCAB
            idx_v[buf + 8, s, sl] = xs + 4 * _VOCAB
            idx_v[buf + 10, s, sl] = ys + 5 * _VOCAB
            idx_v[buf + 12, s, sl] = x1 + 6 * _VOCAB
            idx_v[buf + 14, s, sl] = y1 + 7 * _VOCAB
            idx_v[buf + 16, s, sl] = x3 + 8 * _VOCAB
            idx_v[buf + 18, s, sl] = y3 + 9 * _VOCAB

    def prep(p, buf):
        load_boxes(p, buf)
        compute_idx(buf, 0)
        compute_idx(buf, 1)

    prep(0, 0)

    def drain_outs():
        pltpu.make_async_copy(acc_v.at[0], out_hbm.at[pl.ds(0, _B)],
                              osem_a).wait()
        pltpu.make_async_copy(acc_v.at[1], out_hbm.at[pl.ds(0, _B)],
                              osem_b).wait()

    def pair(p, carry):
        buf = lax.rem(p, 2)
        a0 = 0
        a1 = 1
        base_a = wid * _PER_W + (2 * p) * _B
        base_b = base_a + _B

        @pl.when(p > 0)
        def _():
            drain_outs()

        da = pltpu.async_copy(tab_sp.at[idx_v.at[buf].at[0]], acc_v.at[a0],
                              sem_a)
        db = pltpu.async_copy(tab_sp.at[idx_v.at[buf].at[1]], acc_v.at[a1],
                              sem_b)

        prep(jnp.minimum(p + 1, _NPAIR - 1), 1 - buf)

        for t in range(1, _NT):
            da.wait()
            da = pltpu.async_copy(tab_sp.at[idx_v.at[buf + 2 * t].at[0]],
                                  acc_v.at[a0], sem_a, add=True)
            db.wait()
            db = pltpu.async_copy(tab_sp.at[idx_v.at[buf + 2 * t].at[1]],
                                  acc_v.at[a1], sem_b, add=True)
        da.wait()
        pltpu.async_copy(acc_v.at[a0], out_hbm.at[pl.ds(base_a, _B)], osem_a)
        db.wait()
        pltpu.async_copy(acc_v.at[a1], out_hbm.at[pl.ds(base_b, _B)], osem_b)
        return carry

    lax.fori_loop(0, _NPAIR, pair, 0)
    drain_outs()


def kernel(boxes, w_embed, h_embed, cx_embed, cy_embed, xskew_embed,
           yskew_embed, x1_embed, y1_embed, x3_embed, y3_embed,
           xspecial_embed):
    boxes_t = boxes.astype(jnp.int32).reshape(_N, 6).T

    cx2 = cx_embed + xspecial_embed[0][None, :]
    cx2 = cx2.at[0].add(xspecial_embed[1] - xspecial_embed[0])
    tab = jnp.concatenate(
        [w_embed, h_embed, cx2, cy_embed, xskew_embed, yskew_embed,
         x1_embed, y1_embed, x3_embed, y3_embed], axis=0) * 0.1

    mesh = plsc.VectorSubcoreMesh(core_axis_name="c", subcore_axis_name="s")
    out = pl.kernel(
        _sc_body,
        out_type=jax.ShapeDtypeStruct((_N, _H), jnp.float32),
        mesh=mesh,
        scratch_types=[
            pltpu.VMEM((4, 6, _B), jnp.int32),
            pltpu.VMEM((2 * _NT, 2, _B), jnp.int32),
            pltpu.VMEM((2, _B, _H), jnp.float32),
            pltpu.VMEM_SHARED((_NT * _VOCAB, _H), jnp.float32),
            pltpu.SemaphoreType.DMA,
            pltpu.SemaphoreType.DMA,
            pltpu.SemaphoreType.DMA,
            pltpu.SemaphoreType.DMA,
        ],
    )(boxes_t, tab)

    return out.reshape(boxes.shape[0], boxes.shape[1], _H)
